# baseline (device time: 12713 ns/iter reference)
import jax
import jax.numpy as jnp
from jax import lax
from jax.experimental import pallas as pl
from jax.experimental.pallas import tpu as pltpu

N_CHUNKS = 8


def kernel(A, B):
    m, k = A.shape
    _, n = B.shape
    ck = m // N_CHUNKS

    def body(a_ref, b_ref, out_ref, xsend_buf, xrecv_buf,
             xsend_sems, xrecv_sems):
        my_x = lax.axis_index("x")
        my_y = lax.axis_index("y")
        xpeer = (1 - my_x, my_y)

        barrier_sem = pltpu.get_barrier_semaphore()
        pl.semaphore_signal(
            barrier_sem, inc=1, device_id=xpeer,
            device_id_type=pl.DeviceIdType.MESH,
        )
        pl.semaphore_wait(barrier_sem, 1)

        def x_rdma(c):
            sl = pl.ds(c * ck, ck)
            return pltpu.make_async_remote_copy(
                src_ref=xsend_buf.at[sl],
                dst_ref=xrecv_buf.at[sl],
                send_sem=xsend_sems.at[c],
                recv_sem=xrecv_sems.at[c],
                device_id=xpeer,
                device_id_type=pl.DeviceIdType.MESH,
            )

        for c in range(N_CHUNKS):
            sl = pl.ds(c * ck, ck)
            xsend_buf[sl, :] = jnp.dot(
                a_ref[sl, :].astype(jnp.bfloat16),
                b_ref[...].astype(jnp.bfloat16),
                preferred_element_type=jnp.float32,
            ).astype(jnp.bfloat16)
            x_rdma(c).start()

        for c in range(N_CHUNKS):
            x_rdma(c).wait_recv()
            sl = pl.ds(c * ck, ck)
            out_ref[sl, :] = (
                xsend_buf[sl, :].astype(jnp.float32)
                + xrecv_buf[sl, :].astype(jnp.float32)
            )

        for c in range(N_CHUNKS):
            x_rdma(c).wait_send()

    return pl.pallas_call(
        body,
        out_shape=jax.ShapeDtypeStruct((m, n), jnp.float32),
        in_specs=[
            pl.BlockSpec(memory_space=pltpu.VMEM),
            pl.BlockSpec(memory_space=pltpu.VMEM),
        ],
        out_specs=pl.BlockSpec(memory_space=pltpu.VMEM),
        scratch_shapes=[
            pltpu.VMEM((m, n), jnp.bfloat16),
            pltpu.VMEM((m, n), jnp.bfloat16),
            pltpu.SemaphoreType.DMA((N_CHUNKS,)),
            pltpu.SemaphoreType.DMA((N_CHUNKS,)),
        ],
        compiler_params=pltpu.CompilerParams(collective_id=0),
    )(A, B)
